# Initial kernel scaffold; baseline (speedup 1.0000x reference)
import jax
import jax.numpy as jnp
from jax.experimental import pallas as pl


def kernel(obs, gp_sel_summary, hete_pick, W1, b1, W2, b2, W3, b3, Wc1, bc1, Wc2, bc2, Wc3, bc3):
    raise NotImplementedError("write your pallas kernel here")


# speedup vs baseline: 1.3609x; 1.3609x over previous
import jax
import jax.numpy as jnp
from jax.experimental import pallas as pl
from jax.experimental.pallas import tpu as pltpu

N_TP = 3
N_GP = 5
N_EXP = N_TP * N_GP
RAWOB = 128
D_IN = RAWOB + N_TP
DP = 144
H = 128
N_ACT = 32
NT = 1024
NA = 16
T = NT * NA
BLK = 512


def _dense_body(x_ref, pick_ref, W1_ref, b1_ref, W2_ref, b2_ref, W3_ref,
                b3_ref, Wc1_ref, bc1_ref, Wc2_ref, bc2_ref, Wc3_ref,
                bc3_ref, out_ref):
    x = x_ref[...]
    p = pick_ref[...]
    acc = jnp.zeros((BLK, N_ACT), jnp.float32)
    for e in range(N_EXP):
        h = jnp.tanh(jnp.dot(x, W1_ref[e], preferred_element_type=jnp.float32)
                     + b1_ref[e])
        h = jnp.tanh(jnp.dot(h, W2_ref[e], preferred_element_type=jnp.float32)
                     + b2_ref[e])
        y = jnp.dot(h, W3_ref[e], preferred_element_type=jnp.float32) + b3_ref[e]
        acc = jnp.where(p == e, y, acc)
    hc = jnp.tanh(jnp.dot(x, Wc1_ref[...], preferred_element_type=jnp.float32)
                  + bc1_ref[...])
    hc = jnp.tanh(jnp.dot(hc, Wc2_ref[...], preferred_element_type=jnp.float32)
                  + bc2_ref[...])
    v = jnp.dot(hc, Wc3_ref[...], preferred_element_type=jnp.float32) + bc3_ref[...]
    out_ref[...] = jnp.concatenate([acc, v], axis=1)


def kernel(obs, gp_sel_summary, hete_pick, W1, b1, W2, b2, W3, b3,
           Wc1, bc1, Wc2, bc2, Wc3, bc3):
    x = jnp.concatenate(
        [obs.reshape(T, RAWOB), gp_sel_summary.reshape(T, N_TP),
         jnp.zeros((T, DP - D_IN), jnp.float32)], axis=1)
    pick2 = hete_pick.reshape(T, 1).astype(jnp.int32)
    W1p = jnp.pad(W1, ((0, 0), (0, DP - D_IN), (0, 0)))
    Wc1p = jnp.pad(Wc1, ((0, DP - D_IN), (0, 0)))

    grid = (T // BLK,)
    full = lambda *s: pl.BlockSpec(s, lambda i: (0,) * len(s))
    out = pl.pallas_call(
        _dense_body,
        grid=grid,
        in_specs=[
            pl.BlockSpec((BLK, DP), lambda i: (i, 0)),
            pl.BlockSpec((BLK, 1), lambda i: (i, 0)),
            full(N_EXP, DP, H), full(N_EXP, H), full(N_EXP, H, H),
            full(N_EXP, H), full(N_EXP, H, N_ACT), full(N_EXP, N_ACT),
            full(DP, H), full(H), full(H, H), full(H), full(H, 1), full(1),
        ],
        out_specs=pl.BlockSpec((BLK, N_ACT + 1), lambda i: (i, 0)),
        out_shape=jax.ShapeDtypeStruct((T, N_ACT + 1), jnp.float32),
        compiler_params=pltpu.CompilerParams(
            dimension_semantics=("arbitrary",)),
    )(x, pick2, W1p, b1, W2, b2, W3, b3, Wc1p, bc1, Wc2, bc2, Wc3, bc3)
    return out.reshape(NT, NA, N_ACT + 1)
